# baseline (speedup 1.0000x reference)
import functools

import jax
import jax.numpy as jnp
from jax import lax
from jax.experimental import pallas as pl
from jax.experimental.pallas import tpu as pltpu
from jax.experimental.pallas import tpu_sc as plsc

_CH = 128
_GRP = 8


@functools.lru_cache(maxsize=None)
def _make_gather(N, K, D):
    info = plsc.get_sparse_core_info()
    NC, NS = info.num_cores, info.num_subcores
    NW = NC * NS
    B = N * K
    assert B % (NW * _CH * _GRP) == 0, (B, NW)
    nch = B // (NW * _CH)
    ngrp = nch // _GRP
    grows = _GRP * _CH
    mesh = plsc.VectorSubcoreMesh(core_axis_name="c", subcore_axis_name="s")

    @functools.partial(
        pl.kernel,
        out_type=jax.ShapeDtypeStruct((B, D), jnp.float32),
        mesh=mesh,
        compiler_params=pltpu.CompilerParams(use_tc_tiling_on_sc=False),
        scratch_types=[
            pltpu.VMEM((nch, _CH), jnp.int32),
            pltpu.VMEM((grows, D), jnp.float32),
            pltpu.VMEM((grows, D), jnp.float32),
            pltpu.SemaphoreType.DMA,
            pltpu.SemaphoreType.DMA,
            pltpu.SemaphoreType.DMA,
            pltpu.SemaphoreType.DMA,
        ],
    )
    def gather_k(table_hbm, idx_hbm, out_hbm, idx_v, rows0, rows1,
                 semg0, semg1, semw0, semw1):
        wid = lax.axis_index("s") * NC + lax.axis_index("c")
        r_base = wid * (ngrp * grows)
        pltpu.sync_copy(idx_hbm.at[wid], idx_v)

        bufs = (rows0, rows1)
        semg = (semg0, semg1)
        semw = (semw0, semw1)

        def fire(g, slot):
            return [
                pltpu.async_copy(
                    table_hbm.at[idx_v.at[g * _GRP + j]],
                    bufs[slot].at[pl.ds(j * _CH, _CH)],
                    semg[slot],
                )
                for j in range(_GRP)
            ]

        gdesc = [fire(0, 0), None]
        wdesc = [None, None]
        for g in range(ngrp):
            cur, nxt = g % 2, (g + 1) % 2
            if g + 1 < ngrp:
                if wdesc[nxt] is not None:
                    wdesc[nxt].wait()
                gdesc[nxt] = fire(g + 1, nxt)
            for d in gdesc[cur]:
                d.wait()
            wdesc[cur] = pltpu.async_copy(
                bufs[cur],
                out_hbm.at[pl.ds(r_base + g * grows, grows)],
                semw[cur],
            )
        wdesc[0].wait()
        wdesc[1].wait()

    return gather_k, NW, nch


def _sumsq_body(blk_cols, total_cols, x_ref, o_ref):
    i = pl.program_id(0)

    @pl.when(i == 0)
    def _init():
        o_ref[0, 0] = jnp.float32(0.0)

    x = x_ref[...]
    col = jax.lax.broadcasted_iota(jnp.int32, x.shape, 1) + i * blk_cols
    x = jnp.where(col < total_cols, x, 0.0)
    o_ref[0, 0] += jnp.sum(x * x)


def _sumsq(table_t):
    rows, cols = table_t.shape
    blk = 65536
    nblk = pl.cdiv(cols, blk)
    return pl.pallas_call(
        functools.partial(_sumsq_body, blk, cols),
        grid=(nblk,),
        in_specs=[pl.BlockSpec((rows, blk), lambda i: (0, i))],
        out_specs=pl.BlockSpec(memory_space=pltpu.SMEM),
        out_shape=jax.ShapeDtypeStruct((1, 1), jnp.float32),
    )(table_t)


def kernel(input, kernel):
    table = kernel
    n, k = input.shape
    keys, feat = table.shape
    B = n * k

    gather_k, NW, nch = _make_gather(n, k, feat)
    idx = input.reshape(-1).astype(jnp.int32).reshape(NW, nch, _CH)
    lin = lax.optimization_barrier(table.reshape(-1, 128))
    fetched2d = gather_k(lin.reshape(keys, feat), idx)
    y = lax.optimization_barrier(fetched2d.reshape(n, k * feat))
    fetched = y.reshape(n, k, feat)

    ss = _sumsq(table.T)
    kl = ss[0, 0] * jnp.float32(0.5 / (keys * feat))
    return fetched, kl


# speedup vs baseline: 1.3725x; 1.3725x over previous
import functools

import jax
import jax.numpy as jnp
from jax import lax
from jax.experimental import pallas as pl
from jax.experimental.pallas import tpu as pltpu
from jax.experimental.pallas import tpu_sc as plsc

_CH = 128
_GRP = 8


@functools.lru_cache(maxsize=None)
def _make_gather(N, K, D):
    info = plsc.get_sparse_core_info()
    NC, NS = info.num_cores, info.num_subcores
    NW = NC * NS
    B = N * K
    assert B % (NW * _CH * _GRP) == 0, (B, NW)
    nch = B // (NW * _CH)
    ngrp = nch // _GRP
    grows = _GRP * _CH
    mesh = plsc.VectorSubcoreMesh(core_axis_name="c", subcore_axis_name="s")

    @functools.partial(
        pl.kernel,
        out_type=jax.ShapeDtypeStruct((B, D), jnp.float32),
        mesh=mesh,
        compiler_params=pltpu.CompilerParams(use_tc_tiling_on_sc=False),
        scratch_types=[
            pltpu.VMEM((nch, _CH), jnp.int32),
            pltpu.VMEM((grows, D), jnp.float32),
            pltpu.VMEM((grows, D), jnp.float32),
            pltpu.SemaphoreType.DMA,
            pltpu.SemaphoreType.DMA,
            pltpu.SemaphoreType.DMA,
            pltpu.SemaphoreType.DMA,
        ],
    )
    def gather_k(table_hbm, idx_hbm, out_hbm, idx_v, rows0, rows1,
                 semg0, semg1, semw0, semw1):
        wid = lax.axis_index("s") * NC + lax.axis_index("c")
        r_base = wid * (ngrp * grows)
        pltpu.sync_copy(idx_hbm.at[wid], idx_v)

        bufs = (rows0, rows1)
        semg = (semg0, semg1)
        semw = (semw0, semw1)

        def fire(g, slot):
            return [
                pltpu.async_copy(
                    table_hbm.at[idx_v.at[g * _GRP + j]],
                    bufs[slot].at[pl.ds(j * _CH, _CH)],
                    semg[slot],
                )
                for j in range(_GRP)
            ]

        gdesc = [fire(0, 0), None]
        wdesc = [None, None]
        for g in range(ngrp):
            cur, nxt = g % 2, (g + 1) % 2
            if g + 1 < ngrp:
                if wdesc[nxt] is not None:
                    wdesc[nxt].wait()
                gdesc[nxt] = fire(g + 1, nxt)
            for d in gdesc[cur]:
                d.wait()
            wdesc[cur] = pltpu.async_copy(
                bufs[cur],
                out_hbm.at[pl.ds(r_base + g * grows, grows)],
                semw[cur],
            )
        wdesc[0].wait()
        wdesc[1].wait()

    return gather_k, NW, nch


def _relayout_body(x_ref, o_ref, scratch):
    scratch[...] = x_ref[...].T
    o_ref[...] = jnp.concatenate([scratch[j::4, :] for j in range(4)], axis=1)


def _relayout(table_t):
    feat, keys = table_t.shape
    C = 4096
    nblk = pl.cdiv(keys, C)
    return pl.pallas_call(
        _relayout_body,
        grid=(nblk,),
        in_specs=[pl.BlockSpec((feat, C), lambda i: (0, i))],
        out_specs=pl.BlockSpec((C * feat // 128, 128), lambda i: (i, 0)),
        out_shape=jax.ShapeDtypeStruct((keys * feat // 128, 128), jnp.float32),
        scratch_shapes=[pltpu.VMEM((C, feat), jnp.float32)],
    )(table_t)


def _sumsq_body(blk_cols, total_cols, x_ref, o_ref):
    i = pl.program_id(0)

    @pl.when(i == 0)
    def _init():
        o_ref[0, 0] = jnp.float32(0.0)

    x = x_ref[...]
    col = jax.lax.broadcasted_iota(jnp.int32, x.shape, 1) + i * blk_cols
    x = jnp.where(col < total_cols, x, 0.0)
    o_ref[0, 0] += jnp.sum(x * x)


def _sumsq(table_t):
    rows, cols = table_t.shape
    blk = 65536
    nblk = pl.cdiv(cols, blk)
    return pl.pallas_call(
        functools.partial(_sumsq_body, blk, cols),
        grid=(nblk,),
        in_specs=[pl.BlockSpec((rows, blk), lambda i: (0, i))],
        out_specs=pl.BlockSpec(memory_space=pltpu.SMEM),
        out_shape=jax.ShapeDtypeStruct((1, 1), jnp.float32),
    )(table_t)


def kernel(input, kernel):
    table = kernel
    n, k = input.shape
    keys, feat = table.shape
    B = n * k

    gather_k, NW, nch = _make_gather(n, k, feat)
    idx = input.reshape(-1).astype(jnp.int32).reshape(NW, nch, _CH)
    lin = _relayout(table.T)
    fetched2d = gather_k(lin.reshape(keys, feat), idx)
    y = lax.optimization_barrier(fetched2d.reshape(n, k * feat))
    fetched = y.reshape(n, k, feat)

    ss = _sumsq(table.T)
    kl = ss[0, 0] * jnp.float32(0.5 / (keys * feat))
    return fetched, kl


# speedup vs baseline: 1.4770x; 1.0761x over previous
import functools

import jax
import jax.numpy as jnp
from jax import lax
from jax.experimental import pallas as pl
from jax.experimental.pallas import tpu as pltpu
from jax.experimental.pallas import tpu_sc as plsc

_CH = 128
_GRP = 8


@functools.lru_cache(maxsize=None)
def _make_gather(N, K, D):
    info = plsc.get_sparse_core_info()
    NC, NS = info.num_cores, info.num_subcores
    NW = NC * NS
    B = N * K
    assert B % (NW * _CH * _GRP) == 0, (B, NW)
    nch = B // (NW * _CH)
    ngrp = nch // _GRP
    grows = _GRP * _CH
    mesh = plsc.VectorSubcoreMesh(core_axis_name="c", subcore_axis_name="s")

    @functools.partial(
        pl.kernel,
        out_type=jax.ShapeDtypeStruct((B, D), jnp.float32),
        mesh=mesh,
        compiler_params=pltpu.CompilerParams(use_tc_tiling_on_sc=False),
        scratch_types=[
            pltpu.VMEM((nch, _CH), jnp.int32),
            pltpu.VMEM((grows, D), jnp.float32),
            pltpu.VMEM((grows, D), jnp.float32),
            pltpu.SemaphoreType.DMA,
            pltpu.SemaphoreType.DMA,
            pltpu.SemaphoreType.DMA,
            pltpu.SemaphoreType.DMA,
        ],
    )
    def gather_k(table_hbm, idx_hbm, out_hbm, idx_v, rows0, rows1,
                 semg0, semg1, semw0, semw1):
        wid = lax.axis_index("s") * NC + lax.axis_index("c")
        r_base = wid * (ngrp * grows)
        pltpu.sync_copy(idx_hbm.at[wid], idx_v)

        bufs = (rows0, rows1)
        semg = (semg0, semg1)
        semw = (semw0, semw1)

        def fire(g, slot):
            return [
                pltpu.async_copy(
                    table_hbm.at[idx_v.at[g * _GRP + j]],
                    bufs[slot].at[pl.ds(j * _CH, _CH)],
                    semg[slot],
                )
                for j in range(_GRP)
            ]

        gdesc = [fire(0, 0), None]
        wdesc = [None, None]
        for g in range(ngrp):
            cur, nxt = g % 2, (g + 1) % 2
            if g + 1 < ngrp:
                if wdesc[nxt] is not None:
                    wdesc[nxt].wait()
                gdesc[nxt] = fire(g + 1, nxt)
            for d in gdesc[cur]:
                d.wait()
            wdesc[cur] = pltpu.async_copy(
                bufs[cur],
                out_hbm.at[pl.ds(r_base + g * grows, grows)],
                semw[cur],
            )
        wdesc[0].wait()
        wdesc[1].wait()

    return gather_k, NW, nch


def _relayout_body(blk_cols, total_cols, x_ref, o_ref, ss_ref, scratch):
    i = pl.program_id(0)

    @pl.when(i == 0)
    def _init():
        ss_ref[0, 0] = jnp.float32(0.0)

    x = x_ref[...]
    scratch[...] = x.T
    o_ref[...] = jnp.concatenate([scratch[j::4, :] for j in range(4)], axis=1)
    col = jax.lax.broadcasted_iota(jnp.int32, x.shape, 1) + i * blk_cols
    xm = jnp.where(col < total_cols, x, 0.0)
    ss_ref[0, 0] += jnp.sum(xm * xm)


def _relayout(table_t):
    feat, keys = table_t.shape
    C = 4096
    nblk = pl.cdiv(keys, C)
    return pl.pallas_call(
        functools.partial(_relayout_body, C, keys),
        grid=(nblk,),
        in_specs=[pl.BlockSpec((feat, C), lambda i: (0, i))],
        out_specs=[
            pl.BlockSpec((C * feat // 128, 128), lambda i: (i, 0)),
            pl.BlockSpec(memory_space=pltpu.SMEM),
        ],
        out_shape=[
            jax.ShapeDtypeStruct((keys * feat // 128, 128), jnp.float32),
            jax.ShapeDtypeStruct((1, 1), jnp.float32),
        ],
        scratch_shapes=[pltpu.VMEM((C, feat), jnp.float32)],
    )(table_t)


def kernel(input, kernel):
    table = kernel
    n, k = input.shape
    keys, feat = table.shape
    B = n * k

    gather_k, NW, nch = _make_gather(n, k, feat)
    idx = input.reshape(-1).astype(jnp.int32).reshape(NW, nch, _CH)
    lin, ss = _relayout(table.T)
    fetched2d = gather_k(lin.reshape(keys, feat), idx)
    y = lax.optimization_barrier(fetched2d.reshape(n, k * feat))
    fetched = y.reshape(n, k, feat)

    kl = ss[0, 0] * jnp.float32(0.5 / (keys * feat))
    return fetched, kl
